# baseline (speedup 1.0000x reference)
import functools

import jax
import jax.numpy as jnp
from jax import lax
from jax.experimental import pallas as pl
from jax.experimental.pallas import tpu as pltpu
from jax.experimental.pallas import tpu_sc as plsc

_V = 1000
_VP = 1024
_VA = 896
_VT = _VP - _VA
_N = 51200
_LANES = 16

_NW = 32
_PER_W = _N // _NW
_CHUNK = 32
_STEPS = _PER_W // _CHUNK
_TLC = 80


def _lse_body(table_ref, lse_ref):
    x = table_ref[...]
    m = jnp.max(x, axis=1)
    s = jnp.sum(jnp.exp(x - m[:, None]), axis=1)
    lse_ref[...] = jnp.pad(jnp.log(s) + m, (0, _VP - _V))


def _table_lse(table):
    return pl.pallas_call(
        _lse_body,
        out_shape=jax.ShapeDtypeStruct((_VP,), jnp.float32),
    )(table)


_FBLK = 6400


def _finish_body(tail_ref, x_in_ref, x_out_ref, scratch, sem):
    del x_in_ref
    i = pl.program_id(0)
    scratch[...] = tail_ref[...][:, : _V - _VA]
    copy = pltpu.make_async_copy(
        scratch,
        x_out_ref.at[pl.ds(i * _FBLK, _FBLK), pl.ds(_VA, _V - _VA)], sem)
    copy.start()
    copy.wait()


def _finish(tail, x):
    return pl.pallas_call(
        _finish_body,
        grid=(_N // _FBLK,),
        out_shape=jax.ShapeDtypeStruct((_N, _V), jnp.float32),
        in_specs=[
            pl.BlockSpec((_FBLK, _VT), lambda i: (i, 0)),
            pl.BlockSpec(memory_space=pltpu.MemorySpace.HBM),
        ],
        out_specs=pl.BlockSpec(memory_space=pltpu.MemorySpace.HBM),
        scratch_shapes=[
            pltpu.VMEM((_FBLK, _V - _VA), jnp.float32),
            pltpu.SemaphoreType.DMA,
        ],
        input_output_aliases={1: 0},
    )(tail, x)


def _sc_body(table_hbm, flat_hbm, idx_hbm, tgt_hbm, lse_hbm,
             out_hbm, tail_hbm, psum_hbm,
             idx_v, fi_v, lse_v, tl_v, rows0_v, rows1_v, acc_v,
             sg0, sg1, sw0, sw1, st):
    wid = lax.axis_index("s") * 2 + lax.axis_index("c")
    base = wid * _PER_W

    pltpu.sync_copy(idx_hbm.at[pl.ds(base, _PER_W)], idx_v)
    pltpu.sync_copy(tgt_hbm.at[pl.ds(base, _PER_W)], fi_v)
    pltpu.sync_copy(lse_hbm, lse_v)

    def flatten(i, _):
        o = i * _LANES
        fi_v[pl.ds(o, _LANES)] = (
            idx_v[pl.ds(o, _LANES)] * _V + fi_v[pl.ds(o, _LANES)])
        return 0

    lax.fori_loop(0, _PER_W // _LANES, flatten, 0)

    def tl_start(g, _):
        pltpu.async_copy(
            flat_hbm.at[fi_v.at[pl.ds(g * _TLC, _TLC)]],
            tl_v.at[pl.ds(g * _TLC, _TLC)], st)
        return 0

    lax.fori_loop(0, _PER_W // _TLC, tl_start, 0)

    rows = (rows0_v, rows1_v)
    sg = (sg0, sg1)
    sw = (sw0, sw1)

    def gather_start(c, b):
        pltpu.async_copy(
            table_hbm.at[idx_v.at[pl.ds(c * _CHUNK, _CHUNK)]],
            rows[b], sg[b])

    def gather_wait(c, b):
        pltpu.make_async_copy(
            table_hbm.at[idx_v.at[pl.ds(c * _CHUNK, _CHUNK)]],
            rows[b], sg[b]).wait()

    def write_start(c, b):
        r0 = base + c * _CHUNK
        pltpu.async_copy(
            rows[b].at[:, pl.ds(0, _VA)],
            out_hbm.at[pl.ds(r0, _CHUNK), pl.ds(0, _VA)], sw[b])
        pltpu.async_copy(
            rows[b].at[:, pl.ds(_VA, _VT)],
            tail_hbm.at[pl.ds(r0, _CHUNK)], sw[b])

    def write_wait(c, b):
        r0 = base + c * _CHUNK
        pltpu.make_async_copy(
            rows[b].at[:, pl.ds(0, _VA)],
            out_hbm.at[pl.ds(r0, _CHUNK), pl.ds(0, _VA)], sw[b]).wait()
        pltpu.make_async_copy(
            rows[b].at[:, pl.ds(_VA, _VT)],
            tail_hbm.at[pl.ds(r0, _CHUNK)], sw[b]).wait()

    gather_start(0, 0)

    def outer(o, _):
        for b in range(2):
            c = 2 * o + b
            if b == 0:
                @pl.when(o >= 1)
                def _w():
                    write_wait(c - 1, 1)
                gather_start(c + 1, 1)
            else:
                write_wait(c - 1, 0)

                @pl.when(o < _STEPS // 2 - 1)
                def _w():
                    gather_start(c + 1, 0)
            gather_wait(c, b)
            write_start(c, b)
        return 0

    lax.fori_loop(0, _STEPS // 2, outer, 0)

    def tl_drain(g, _):
        pltpu.make_async_copy(
            flat_hbm.at[fi_v.at[pl.ds(g * _TLC, _TLC)]],
            tl_v.at[pl.ds(g * _TLC, _TLC)], st).wait()
        return 0

    lax.fori_loop(0, _PER_W // _TLC, tl_drain, 0)

    def loss_grp(j, acc):
        o = j * _LANES
        iv16 = idx_v[pl.ds(o, _LANES)]
        for k in range(_LANES):
            lse16 = lse_v[pl.ds(iv16[k], _LANES)]
            tl16 = tl_v[pl.ds(o + k, _LANES)]
            acc = acc + (lse16 - tl16)
        return acc

    zero = jnp.zeros((_LANES,), jnp.float32)
    acc = lax.fori_loop(0, _PER_W // _LANES, loss_grp, zero)
    write_wait(_STEPS - 1, 1)
    lane = lax.broadcasted_iota(jnp.int32, (_LANES,), 0)
    acc_v[...] = jnp.where(lane == 0, acc, zero)
    pltpu.sync_copy(acc_v, psum_hbm.at[pl.ds(wid * _LANES, _LANES)])


@functools.partial(
    pl.kernel,
    mesh=plsc.VectorSubcoreMesh(core_axis_name="c", subcore_axis_name="s"),
    out_type=[
        jax.ShapeDtypeStruct((_N, _V), jnp.float32),
        jax.ShapeDtypeStruct((_N, _VT), jnp.float32),
        jax.ShapeDtypeStruct((_NW * _LANES,), jnp.float32),
    ],
    scratch_types=[
        pltpu.VMEM((_PER_W,), jnp.int32),
        pltpu.VMEM((_PER_W,), jnp.int32),
        pltpu.VMEM((_VP,), jnp.float32),
        pltpu.VMEM((_PER_W + _LANES,), jnp.float32),
        pltpu.VMEM((_CHUNK, _VP), jnp.float32),
        pltpu.VMEM((_CHUNK, _VP), jnp.float32),
        pltpu.VMEM((_LANES,), jnp.float32),
        pltpu.SemaphoreType.DMA,
        pltpu.SemaphoreType.DMA,
        pltpu.SemaphoreType.DMA,
        pltpu.SemaphoreType.DMA,
        pltpu.SemaphoreType.DMA,
    ],
)
def _sc_gather(*args):
    _sc_body(*args)


def kernel(idx, targets, table):
    idx_f = idx.reshape(_N).astype(jnp.int32)
    tgt_f = targets.reshape(_N).astype(jnp.int32)
    lse = _table_lse(table)
    table_pad = jnp.pad(table, ((0, 0), (0, _VP - _V)))
    flat = table.reshape(_V * _V)
    x, tail, psum = _sc_gather(table_pad, flat, idx_f, tgt_f, lse)
    logits = _finish(tail, x)
    loss = jnp.sum(psum) * (1.0 / _N)
    return logits, loss


# speedup vs baseline: 1.1558x; 1.1558x over previous
import functools

import jax
import jax.numpy as jnp
from jax import lax
from jax.experimental import pallas as pl
from jax.experimental.pallas import tpu as pltpu
from jax.experimental.pallas import tpu_sc as plsc

_V = 1000
_VP = 1024
_N = 51200
_LANES = 16

_NS = 2
_SN = _N // _NS
_NW = 32
_SPW = _SN // _NW
_CHUNK = 40
_STEPS = _SPW // _CHUNK
_TLC = 80

_TBLK = 512
_TGRID = _SN // _TBLK


def _lse_body(table_ref, lse_ref):
    x = table_ref[...]
    m = jnp.max(x, axis=1)
    s = jnp.sum(jnp.exp(x - m[:, None]), axis=1)
    lse_ref[...] = jnp.pad(jnp.log(s) + m, (0, _VP - _V))


def _table_lse(table):
    return pl.pallas_call(
        _lse_body,
        out_shape=jax.ShapeDtypeStruct((_VP,), jnp.float32),
    )(table)


def _tx_first_body(slab_ref, x_ref):
    x_ref[...] = slab_ref[...][:, :_V].T


def _tx_body(slab_ref, xprev_ref, x_ref):
    del xprev_ref
    x_ref[...] = slab_ref[...][:, :_V].T


def _transpose_stage(s, slab, x):
    out_shape = jax.ShapeDtypeStruct((_V, _N), jnp.float32)
    slab_spec = pl.BlockSpec((_TBLK, _VP), lambda i: (i, 0))
    x_spec = pl.BlockSpec((_V, _TBLK), lambda i, s=s: (0, s * _TGRID + i))
    if x is None:
        return pl.pallas_call(
            _tx_first_body,
            grid=(_TGRID,),
            out_shape=out_shape,
            in_specs=[slab_spec],
            out_specs=x_spec,
        )(slab)
    return pl.pallas_call(
        _tx_body,
        grid=(_TGRID,),
        out_shape=out_shape,
        in_specs=[slab_spec, pl.BlockSpec(memory_space=pltpu.MemorySpace.HBM)],
        out_specs=x_spec,
        input_output_aliases={1: 0},
    )(slab, x)


def _sc_body(stage, table_hbm, flat_hbm, idx_hbm, tgt_hbm, lse_hbm,
             slab_hbm, psum_hbm,
             idx_v, fi_v, lse_v, tl_v, rows0_v, rows1_v, acc_v,
             sg0, sg1, sw0, sw1, st):
    wid = lax.axis_index("s") * 2 + lax.axis_index("c")
    base = stage * _SN + wid * _SPW
    lbase = wid * _SPW

    pltpu.sync_copy(idx_hbm.at[pl.ds(base, _SPW)], idx_v)
    pltpu.sync_copy(tgt_hbm.at[pl.ds(base, _SPW)], fi_v)
    pltpu.sync_copy(lse_hbm, lse_v)

    def flatten(i, _):
        o = i * _LANES
        fi_v[pl.ds(o, _LANES)] = (
            idx_v[pl.ds(o, _LANES)] * _V + fi_v[pl.ds(o, _LANES)])
        return 0

    lax.fori_loop(0, _SPW // _LANES, flatten, 0)

    def tl_start(g, _):
        pltpu.async_copy(
            flat_hbm.at[fi_v.at[pl.ds(g * _TLC, _TLC)]],
            tl_v.at[pl.ds(g * _TLC, _TLC)], st)
        return 0

    lax.fori_loop(0, _SPW // _TLC, tl_start, 0)

    rows = (rows0_v, rows1_v)
    sg = (sg0, sg1)
    sw = (sw0, sw1)

    def gather_start(c, b):
        pltpu.async_copy(
            table_hbm.at[idx_v.at[pl.ds(c * _CHUNK, _CHUNK)]],
            rows[b], sg[b])

    def gather_wait(c, b):
        pltpu.make_async_copy(
            table_hbm.at[idx_v.at[pl.ds(c * _CHUNK, _CHUNK)]],
            rows[b], sg[b]).wait()

    def write_start(c, b):
        pltpu.async_copy(
            rows[b], slab_hbm.at[pl.ds(lbase + c * _CHUNK, _CHUNK)], sw[b])

    def write_wait(c, b):
        pltpu.make_async_copy(
            rows[b], slab_hbm.at[pl.ds(lbase + c * _CHUNK, _CHUNK)],
            sw[b]).wait()

    gather_start(0, 0)

    def outer(o, _):
        for b in range(2):
            c = 2 * o + b
            if b == 0:
                @pl.when(o >= 1)
                def _w():
                    write_wait(c - 1, 1)
                gather_start(c + 1, 1)
            else:
                write_wait(c - 1, 0)

                @pl.when(o < _STEPS // 2 - 1)
                def _w():
                    gather_start(c + 1, 0)
            gather_wait(c, b)
            write_start(c, b)
        return 0

    lax.fori_loop(0, _STEPS // 2, outer, 0)

    def tl_drain(g, _):
        pltpu.make_async_copy(
            flat_hbm.at[fi_v.at[pl.ds(g * _TLC, _TLC)]],
            tl_v.at[pl.ds(g * _TLC, _TLC)], st).wait()
        return 0

    lax.fori_loop(0, _SPW // _TLC, tl_drain, 0)

    def loss_grp(j, acc):
        o = j * _LANES
        iv16 = idx_v[pl.ds(o, _LANES)]
        for k in range(_LANES):
            lse16 = lse_v[pl.ds(iv16[k], _LANES)]
            tl16 = tl_v[pl.ds(o + k, _LANES)]
            acc = acc + (lse16 - tl16)
        return acc

    zero = jnp.zeros((_LANES,), jnp.float32)
    acc = lax.fori_loop(0, _SPW // _LANES, loss_grp, zero)
    write_wait(_STEPS - 1, 1)
    lane = lax.broadcasted_iota(jnp.int32, (_LANES,), 0)
    acc_v[...] = jnp.where(lane == 0, acc, zero)
    pltpu.sync_copy(acc_v, psum_hbm.at[pl.ds(wid * _LANES, _LANES)])


def _make_sc_stage(stage):
    @functools.partial(
        pl.kernel,
        mesh=plsc.VectorSubcoreMesh(
            core_axis_name="c", subcore_axis_name="s"),
        out_type=[
            jax.ShapeDtypeStruct((_SN, _VP), jnp.float32),
            jax.ShapeDtypeStruct((_NW * _LANES,), jnp.float32),
        ],
        scratch_types=[
            pltpu.VMEM((_SPW,), jnp.int32),
            pltpu.VMEM((_SPW,), jnp.int32),
            pltpu.VMEM((_VP,), jnp.float32),
            pltpu.VMEM((_SPW + _LANES,), jnp.float32),
            pltpu.VMEM((_CHUNK, _VP), jnp.float32),
            pltpu.VMEM((_CHUNK, _VP), jnp.float32),
            pltpu.VMEM((_LANES,), jnp.float32),
            pltpu.SemaphoreType.DMA,
            pltpu.SemaphoreType.DMA,
            pltpu.SemaphoreType.DMA,
            pltpu.SemaphoreType.DMA,
            pltpu.SemaphoreType.DMA,
        ],
        name=f"sc_gather_s{stage}",
    )
    def _stage_kernel(*args):
        _sc_body(stage, *args)

    return _stage_kernel


_SC_STAGES = [_make_sc_stage(s) for s in range(_NS)]


def kernel(idx, targets, table):
    idx_f = idx.reshape(_N).astype(jnp.int32)
    tgt_f = targets.reshape(_N).astype(jnp.int32)
    lse = _table_lse(table)
    table_pad = jnp.pad(table, ((0, 0), (0, _VP - _V)))
    flat = table.reshape(_V * _V)

    psums = []
    x = None
    for s2 in range(_NS):
        slab, psum = _SC_STAGES[s2](table_pad, flat, idx_f, tgt_f, lse)
        psums.append(psum)
        x = _transpose_stage(s2, slab, x)
    loss = sum(jnp.sum(p) for p in psums) * (1.0 / _N)
    return x.T, loss


# speedup vs baseline: 1.1825x; 1.0231x over previous
import functools

import jax
import jax.numpy as jnp
from jax import lax
from jax.experimental import pallas as pl
from jax.experimental.pallas import tpu as pltpu
from jax.experimental.pallas import tpu_sc as plsc

_V = 1000
_VP = 1024
_N = 51200
_LANES = 16

_NS = 1
_SN = _N // _NS
_NW = 32
_SPW = _SN // _NW
_CHUNK = 40
_STEPS = _SPW // _CHUNK
_TLC = 80

_TBLK = 512
_TGRID = _SN // _TBLK


def _lse_body(table_ref, lse_ref):
    x = table_ref[...]
    m = jnp.max(x, axis=1)
    s = jnp.sum(jnp.exp(x - m[:, None]), axis=1)
    lse_ref[...] = jnp.pad(jnp.log(s) + m, (0, _VP - _V))


def _table_lse(table):
    return pl.pallas_call(
        _lse_body,
        out_shape=jax.ShapeDtypeStruct((_VP,), jnp.float32),
    )(table)


def _tx_first_body(slab_ref, x_ref):
    x_ref[...] = slab_ref[...][:, :_V].T


def _tx_body(slab_ref, xprev_ref, x_ref):
    del xprev_ref
    x_ref[...] = slab_ref[...][:, :_V].T


def _transpose_stage(s, slab, x):
    out_shape = jax.ShapeDtypeStruct((_V, _N), jnp.float32)
    slab_spec = pl.BlockSpec((_TBLK, _VP), lambda i: (i, 0))
    x_spec = pl.BlockSpec((_V, _TBLK), lambda i, s=s: (0, s * _TGRID + i))
    if x is None:
        return pl.pallas_call(
            _tx_first_body,
            grid=(_TGRID,),
            out_shape=out_shape,
            in_specs=[slab_spec],
            out_specs=x_spec,
        )(slab)
    return pl.pallas_call(
        _tx_body,
        grid=(_TGRID,),
        out_shape=out_shape,
        in_specs=[slab_spec, pl.BlockSpec(memory_space=pltpu.MemorySpace.HBM)],
        out_specs=x_spec,
        input_output_aliases={1: 0},
    )(slab, x)


def _sc_body(stage, table_hbm, flat_hbm, idx_hbm, tgt_hbm, lse_hbm,
             slab_hbm, psum_hbm,
             idx_v, fi_v, lse_v, tl_v, rows0_v, rows1_v, acc_v,
             sg0, sg1, sw0, sw1, st):
    wid = lax.axis_index("s") * 2 + lax.axis_index("c")
    base = stage * _SN + wid * _SPW
    lbase = wid * _SPW

    pltpu.sync_copy(idx_hbm.at[pl.ds(base, _SPW)], idx_v)
    pltpu.sync_copy(tgt_hbm.at[pl.ds(base, _SPW)], fi_v)
    pltpu.sync_copy(lse_hbm, lse_v)

    def flatten(i, _):
        o = i * _LANES
        fi_v[pl.ds(o, _LANES)] = (
            idx_v[pl.ds(o, _LANES)] * _V + fi_v[pl.ds(o, _LANES)])
        return 0

    lax.fori_loop(0, _SPW // _LANES, flatten, 0)

    def tl_start(g, _):
        pltpu.async_copy(
            flat_hbm.at[fi_v.at[pl.ds(g * _TLC, _TLC)]],
            tl_v.at[pl.ds(g * _TLC, _TLC)], st)
        return 0

    lax.fori_loop(0, _SPW // _TLC, tl_start, 0)

    rows = (rows0_v, rows1_v)
    sg = (sg0, sg1)
    sw = (sw0, sw1)

    def gather_start(c, b):
        pltpu.async_copy(
            table_hbm.at[idx_v.at[pl.ds(c * _CHUNK, _CHUNK)]],
            rows[b], sg[b])

    def gather_wait(c, b):
        pltpu.make_async_copy(
            table_hbm.at[idx_v.at[pl.ds(c * _CHUNK, _CHUNK)]],
            rows[b], sg[b]).wait()

    def write_start(c, b):
        pltpu.async_copy(
            rows[b], slab_hbm.at[pl.ds(lbase + c * _CHUNK, _CHUNK)], sw[b])

    def write_wait(c, b):
        pltpu.make_async_copy(
            rows[b], slab_hbm.at[pl.ds(lbase + c * _CHUNK, _CHUNK)],
            sw[b]).wait()

    gather_start(0, 0)

    def outer(o, _):
        for b in range(2):
            c = 2 * o + b
            if b == 0:
                @pl.when(o >= 1)
                def _w():
                    write_wait(c - 1, 1)
                gather_start(c + 1, 1)
            else:
                write_wait(c - 1, 0)

                @pl.when(o < _STEPS // 2 - 1)
                def _w():
                    gather_start(c + 1, 0)
            gather_wait(c, b)
            write_start(c, b)
        return 0

    lax.fori_loop(0, _STEPS // 2, outer, 0)

    def tl_drain(g, _):
        pltpu.make_async_copy(
            flat_hbm.at[fi_v.at[pl.ds(g * _TLC, _TLC)]],
            tl_v.at[pl.ds(g * _TLC, _TLC)], st).wait()
        return 0

    lax.fori_loop(0, _SPW // _TLC, tl_drain, 0)

    def loss_grp(j, acc):
        o = j * _LANES
        iv16 = idx_v[pl.ds(o, _LANES)]
        for k in range(_LANES):
            lse16 = lse_v[pl.ds(iv16[k], _LANES)]
            tl16 = tl_v[pl.ds(o + k, _LANES)]
            acc = acc + (lse16 - tl16)
        return acc

    zero = jnp.zeros((_LANES,), jnp.float32)
    acc = lax.fori_loop(0, _SPW // _LANES, loss_grp, zero)
    write_wait(_STEPS - 1, 1)
    lane = lax.broadcasted_iota(jnp.int32, (_LANES,), 0)
    acc_v[...] = jnp.where(lane == 0, acc, zero)
    pltpu.sync_copy(acc_v, psum_hbm.at[pl.ds(wid * _LANES, _LANES)])


def _make_sc_stage(stage):
    @functools.partial(
        pl.kernel,
        mesh=plsc.VectorSubcoreMesh(
            core_axis_name="c", subcore_axis_name="s"),
        out_type=[
            jax.ShapeDtypeStruct((_SN, _VP), jnp.float32),
            jax.ShapeDtypeStruct((_NW * _LANES,), jnp.float32),
        ],
        scratch_types=[
            pltpu.VMEM((_SPW,), jnp.int32),
            pltpu.VMEM((_SPW,), jnp.int32),
            pltpu.VMEM((_VP,), jnp.float32),
            pltpu.VMEM((_SPW + _LANES,), jnp.float32),
            pltpu.VMEM((_CHUNK, _VP), jnp.float32),
            pltpu.VMEM((_CHUNK, _VP), jnp.float32),
            pltpu.VMEM((_LANES,), jnp.float32),
            pltpu.SemaphoreType.DMA,
            pltpu.SemaphoreType.DMA,
            pltpu.SemaphoreType.DMA,
            pltpu.SemaphoreType.DMA,
            pltpu.SemaphoreType.DMA,
        ],
        name=f"sc_gather_s{stage}",
    )
    def _stage_kernel(*args):
        _sc_body(stage, *args)

    return _stage_kernel


_SC_STAGES = [_make_sc_stage(s) for s in range(_NS)]


def kernel(idx, targets, table):
    idx_f = idx.reshape(_N).astype(jnp.int32)
    tgt_f = targets.reshape(_N).astype(jnp.int32)
    lse = _table_lse(table)
    table_pad = jnp.pad(table, ((0, 0), (0, _VP - _V)))
    flat = table.reshape(_V * _V)

    slab, psum = _SC_STAGES[0](table_pad, flat, idx_f, tgt_f, lse)
    loss = jnp.sum(psum) * (1.0 / _N)
    return slab[:, :_V], loss
